# Initial kernel scaffold; baseline (speedup 1.0000x reference)
import functools

import jax
import jax.numpy as jnp
from jax import lax
from jax.experimental import pallas as pl
from jax.experimental.pallas import tpu as pltpu
from jax.experimental.pallas import tpu_sc as plsc

N = 10000
E = 320000
DH = 128
DOUT = 64
G = 64
NC, NS = 2, 16
NW = NC * NS
LANES = 128
ECP = 2560
EPAD = ECP * LANES
NP = 10016
RPT = NP // NS
CPT = ECP // NW

_MESH = plsc.VectorSubcoreMesh(core_axis_name="c", subcore_axis_name="s")
_HIGH = lax.Precision.HIGHEST


def _deg_body(dst_hbm, zeros_hbm, out_hbm, idx_v, ones_v, hist_sh, sem):
    cid = lax.axis_index("c")
    sid = lax.axis_index("s")
    wid = sid * NC + cid

    def fill(i, carry):
        ones_v[i, :] = jnp.full((16,), 1.0, jnp.float32)
        return carry

    lax.fori_loop(0, LANES, fill, 0)
    pltpu.sync_copy(zeros_hbm.at[pl.ds(sid * RPT, RPT)],
                    hist_sh.at[pl.ds(sid * RPT, RPT)])
    plsc.subcore_barrier()

    def body(i, carry):
        c = wid + i * NW
        pltpu.sync_copy(dst_hbm.at[c], idx_v)
        pltpu.sync_copy(ones_v, hist_sh.at[idx_v], add=True)
        return carry

    lax.fori_loop(0, CPT, body, 0)
    plsc.subcore_barrier()
    pltpu.sync_copy(hist_sh.at[pl.ds(sid * RPT, RPT)],
                    out_hbm.at[cid, pl.ds(sid * RPT, RPT)])


_deg_call = functools.partial(
    pl.kernel,
    out_type=jax.ShapeDtypeStruct((NC, NP, 16), jnp.float32),
    mesh=_MESH,
    scratch_types=[
        pltpu.VMEM((LANES,), jnp.int32),
        pltpu.VMEM((LANES, 16), jnp.float32),
        pltpu.VMEM_SHARED((NP, 16), jnp.float32),
        pltpu.SemaphoreType.DMA,
    ],
)(_deg_body)


def _prop_body(hs_hbm, src_hbm, dst_hbm, zeros_hbm, out_hbm,
               sidx_v, didx_v, rows_v, acc_sh, sem):
    cid = lax.axis_index("c")
    sid = lax.axis_index("s")
    wid = sid * NC + cid

    pltpu.sync_copy(zeros_hbm.at[pl.ds(sid * RPT, RPT)],
                    acc_sh.at[pl.ds(sid * RPT, RPT)])
    plsc.subcore_barrier()

    def body(i, carry):
        c = wid + i * NW
        pltpu.sync_copy(src_hbm.at[c], sidx_v)
        pltpu.sync_copy(dst_hbm.at[c], didx_v)
        pltpu.async_copy(hs_hbm.at[sidx_v], rows_v, sem).wait()
        pltpu.sync_copy(rows_v, acc_sh.at[didx_v], add=True)
        return carry

    lax.fori_loop(0, CPT, body, 0)
    plsc.subcore_barrier()
    pltpu.sync_copy(acc_sh.at[pl.ds(sid * RPT, RPT)],
                    out_hbm.at[cid, pl.ds(sid * RPT, RPT)])


_prop_call = functools.partial(
    pl.kernel,
    out_type=jax.ShapeDtypeStruct((NC, NP, DH), jnp.float32),
    mesh=_MESH,
    scratch_types=[
        pltpu.VMEM((LANES,), jnp.int32),
        pltpu.VMEM((LANES,), jnp.int32),
        pltpu.VMEM((LANES, DH), jnp.float32),
        pltpu.VMEM_SHARED((NP, DH), jnp.float32),
        pltpu.SemaphoreType.DMA,
    ],
)(_prop_body)


def _prep_body(x_ref, w_ref, degp_ref, hs_ref, dinv_ref):
    deg = 1.0 + degp_ref[0, pl.ds(0, N), 0:1] + degp_ref[1, pl.ds(0, N), 0:1]
    dinv = lax.rsqrt(deg)
    h = jnp.dot(x_ref[...], w_ref[...], precision=_HIGH,
                preferred_element_type=jnp.float32)
    hs_ref[pl.ds(0, N), :] = h * dinv
    hs_ref[pl.ds(N, NP - N), :] = jnp.zeros((NP - N, DH), jnp.float32)
    dinv_ref[...] = dinv


def _mid_body(accp_ref, hs_ref, dinv_ref, b_ref, w_ref, out_ref):
    dinv = dinv_ref[...]
    agg = (accp_ref[0, pl.ds(0, N), :] + accp_ref[1, pl.ds(0, N), :]
           + hs_ref[pl.ds(0, N), :])
    z = jnp.maximum(dinv * agg + b_ref[...], 0.0)
    h2 = jnp.dot(z, w_ref[...], precision=_HIGH,
                 preferred_element_type=jnp.float32)
    out_ref[pl.ds(0, N), :] = h2 * dinv
    out_ref[pl.ds(N, NP - N), :] = jnp.zeros((NP - N, DH), jnp.float32)


def _fin_body(accp_ref, hs_ref, dinv_ref, b_ref, batch_ref, wfc_ref, bfc_ref,
              out_ref):
    dinv = dinv_ref[...]
    agg = (accp_ref[0, pl.ds(0, N), :] + accp_ref[1, pl.ds(0, N), :]
           + hs_ref[pl.ds(0, N), :])
    z = jnp.maximum(dinv * agg + b_ref[...], 0.0)
    gid = lax.broadcasted_iota(jnp.int32, (G, N), 0)
    onehot = (batch_ref[...] == gid).astype(jnp.float32)
    sums = jnp.dot(onehot, z, precision=_HIGH,
                   preferred_element_type=jnp.float32)
    cnts = jnp.sum(onehot, axis=1, keepdims=True)
    gpool = sums / jnp.maximum(cnts, 1.0)
    out_ref[...] = jnp.dot(gpool, wfc_ref[...], precision=_HIGH,
                           preferred_element_type=jnp.float32) + bfc_ref[...]


def _tc_call(body, out_shape):
    return pl.pallas_call(body, out_shape=out_shape)


def kernel(x, edge_index, batch, W1, b1, W2, b2, Wfc, bfc):
    src = edge_index[0].astype(jnp.int32)
    dst = edge_index[1].astype(jnp.int32)
    pad = jnp.full((EPAD - E,), N, jnp.int32)
    src2d = jnp.concatenate([src, pad]).reshape(ECP, LANES)
    dst2d = jnp.concatenate([dst, pad]).reshape(ECP, LANES)
    batch2d = batch.astype(jnp.int32).reshape(1, N)
    z128 = jnp.zeros((NP, DH), jnp.float32)
    z16 = jnp.zeros((NP, 16), jnp.float32)
    b1r = b1.reshape(1, DH)
    b2r = b2.reshape(1, DH)
    bfcr = bfc.reshape(1, DOUT)

    degp = _deg_call(dst2d, z16)

    hs1, dinv = _tc_call(
        _prep_body,
        (jax.ShapeDtypeStruct((NP, DH), jnp.float32),
         jax.ShapeDtypeStruct((N, 1), jnp.float32)),
    )(x, W1, degp)

    acc1 = _prop_call(hs1, src2d, dst2d, z128)

    hs2 = _tc_call(
        _mid_body, jax.ShapeDtypeStruct((NP, DH), jnp.float32),
    )(acc1, hs1, dinv, b1r, W2)

    acc2 = _prop_call(hs2, src2d, dst2d, z128)

    out = _tc_call(
        _fin_body, jax.ShapeDtypeStruct((G, DOUT), jnp.float32),
    )(acc2, hs2, dinv, b2r, batch2d, Wfc, bfcr)
    return out


# speedup vs baseline: 8.4990x; 8.4990x over previous
import functools

import jax
import jax.numpy as jnp
from jax import lax
from jax.experimental import pallas as pl
from jax.experimental.pallas import tpu as pltpu
from jax.experimental.pallas import tpu_sc as plsc

N = 10000
E = 320000
DH = 128
DOUT = 64
G = 64
NC, NS = 2, 16
NW = NC * NS
LANES = 128
ECP = 2560
EPAD = ECP * LANES
NP = 10112
RPT = NP // NS
CPT = ECP // NW

_HIGH = lax.Precision.HIGHEST


@functools.cache
def _mesh():
    return plsc.VectorSubcoreMesh(core_axis_name="c", subcore_axis_name="s")


def _deg_body(dst_hbm, zeros_hbm, out_hbm, idx_v, ones_v, hist_sh, sem):
    cid = lax.axis_index("c")
    sid = lax.axis_index("s")
    wid = sid * NC + cid

    def fill(i, carry):
        ones_v[i, :] = jnp.full((16,), 1.0, jnp.float32)
        return carry

    lax.fori_loop(0, LANES, fill, 0)
    pltpu.sync_copy(zeros_hbm.at[pl.ds(sid * RPT, RPT)],
                    hist_sh.at[pl.ds(sid * RPT, RPT)])
    plsc.subcore_barrier()

    def body(i, carry):
        c = wid + i * NW
        pltpu.sync_copy(dst_hbm.at[c], idx_v)
        pltpu.sync_copy(ones_v, hist_sh.at[idx_v], add=True)
        return carry

    lax.fori_loop(0, CPT, body, 0)
    plsc.subcore_barrier()
    pltpu.sync_copy(hist_sh.at[pl.ds(sid * RPT, RPT)],
                    out_hbm.at[cid, pl.ds(sid * RPT, RPT)])


@functools.cache
def _deg_call():
    return pl.kernel(
        _deg_body,
        out_type=jax.ShapeDtypeStruct((NC, NP, 16), jnp.float32),
        mesh=_mesh(),
        scratch_types=[
            pltpu.VMEM((LANES,), jnp.int32),
            pltpu.VMEM((LANES, 16), jnp.float32),
            pltpu.VMEM_SHARED((NP, 16), jnp.float32),
            pltpu.SemaphoreType.DMA,
        ],
    )


def _prop_body(hs_hbm, src_hbm, dst_hbm, zeros_hbm, out_hbm,
               sidx_v, didx_v, rows_v, acc_sh, sem):
    cid = lax.axis_index("c")
    sid = lax.axis_index("s")
    wid = sid * NC + cid

    pltpu.sync_copy(zeros_hbm.at[pl.ds(sid * RPT, RPT)],
                    acc_sh.at[pl.ds(sid * RPT, RPT)])
    plsc.subcore_barrier()

    def body(i, carry):
        c = wid + i * NW
        pltpu.sync_copy(src_hbm.at[c], sidx_v)
        pltpu.sync_copy(dst_hbm.at[c], didx_v)
        pltpu.async_copy(hs_hbm.at[sidx_v], rows_v, sem).wait()
        pltpu.sync_copy(rows_v, acc_sh.at[didx_v], add=True)
        return carry

    lax.fori_loop(0, CPT, body, 0)
    plsc.subcore_barrier()
    pltpu.sync_copy(acc_sh.at[pl.ds(sid * RPT, RPT)],
                    out_hbm.at[cid, pl.ds(sid * RPT, RPT)])


@functools.cache
def _prop_call():
    return pl.kernel(
        _prop_body,
        out_type=jax.ShapeDtypeStruct((NC, NP, DH), jnp.float32),
        mesh=_mesh(),
        scratch_types=[
            pltpu.VMEM((LANES,), jnp.int32),
            pltpu.VMEM((LANES,), jnp.int32),
            pltpu.VMEM((LANES, DH), jnp.float32),
            pltpu.VMEM_SHARED((NP, DH), jnp.float32),
            pltpu.SemaphoreType.DMA,
        ],
    )


def _prep_body(x_ref, w_ref, degp_ref, hs_ref, dinv_ref):
    deg = 1.0 + degp_ref[0, pl.ds(0, N), 0:1] + degp_ref[1, pl.ds(0, N), 0:1]
    dinv = lax.rsqrt(deg)
    h = jnp.dot(x_ref[...], w_ref[...], precision=_HIGH,
                preferred_element_type=jnp.float32)
    hs_ref[pl.ds(0, N), :] = h * dinv
    hs_ref[pl.ds(N, NP - N), :] = jnp.zeros((NP - N, DH), jnp.float32)
    dinv_ref[...] = dinv


def _mid_body(accp_ref, hs_ref, dinv_ref, b_ref, w_ref, out_ref):
    dinv = dinv_ref[...]
    agg = (accp_ref[0, pl.ds(0, N), :] + accp_ref[1, pl.ds(0, N), :]
           + hs_ref[pl.ds(0, N), :])
    z = jnp.maximum(dinv * agg + b_ref[...], 0.0)
    h2 = jnp.dot(z, w_ref[...], precision=_HIGH,
                 preferred_element_type=jnp.float32)
    out_ref[pl.ds(0, N), :] = h2 * dinv
    out_ref[pl.ds(N, NP - N), :] = jnp.zeros((NP - N, DH), jnp.float32)


def _fin_body(accp_ref, hs_ref, dinv_ref, b_ref, batch_ref, wfc_ref, bfc_ref,
              out_ref):
    dinv = dinv_ref[...]
    agg = (accp_ref[0, pl.ds(0, N), :] + accp_ref[1, pl.ds(0, N), :]
           + hs_ref[pl.ds(0, N), :])
    z = jnp.maximum(dinv * agg + b_ref[...], 0.0)
    gid = lax.broadcasted_iota(jnp.int32, (G, N), 0)
    onehot = (batch_ref[...] == gid).astype(jnp.float32)
    sums = jnp.dot(onehot, z, precision=_HIGH,
                   preferred_element_type=jnp.float32)
    cnts = jnp.sum(onehot, axis=1, keepdims=True)
    gpool = sums / jnp.maximum(cnts, 1.0)
    out_ref[...] = jnp.dot(gpool, wfc_ref[...], precision=_HIGH,
                           preferred_element_type=jnp.float32) + bfc_ref[...]


def _tc_call(body, out_shape):
    return pl.pallas_call(body, out_shape=out_shape)


def kernel(x, edge_index, batch, W1, b1, W2, b2, Wfc, bfc):
    src = edge_index[0].astype(jnp.int32)
    dst = edge_index[1].astype(jnp.int32)
    pad = jnp.full((EPAD - E,), N, jnp.int32)
    src2d = jnp.concatenate([src, pad]).reshape(ECP, LANES)
    dst2d = jnp.concatenate([dst, pad]).reshape(ECP, LANES)
    batch2d = batch.astype(jnp.int32).reshape(1, N)
    z128 = jnp.zeros((NP, DH), jnp.float32)
    z16 = jnp.zeros((NP, 16), jnp.float32)
    b1r = b1.reshape(1, DH)
    b2r = b2.reshape(1, DH)
    bfcr = bfc.reshape(1, DOUT)

    degp = _deg_call()(dst2d, z16)

    hs1, dinv = _tc_call(
        _prep_body,
        (jax.ShapeDtypeStruct((NP, DH), jnp.float32),
         jax.ShapeDtypeStruct((N, 1), jnp.float32)),
    )(x, W1, degp)

    acc1 = _prop_call()(hs1, src2d, dst2d, z128)

    hs2 = _tc_call(
        _mid_body, jax.ShapeDtypeStruct((NP, DH), jnp.float32),
    )(acc1, hs1, dinv, b1r, W2)

    acc2 = _prop_call()(hs2, src2d, dst2d, z128)

    out = _tc_call(
        _fin_body, jax.ShapeDtypeStruct((G, DOUT), jnp.float32),
    )(acc2, hs2, dinv, b2r, batch2d, Wfc, bfcr)
    return out


# speedup vs baseline: 9.9281x; 1.1681x over previous
import functools

import jax
import jax.numpy as jnp
from jax import lax
from jax.experimental import pallas as pl
from jax.experimental.pallas import tpu as pltpu
from jax.experimental.pallas import tpu_sc as plsc

N = 10000
E = 320000
DH = 128
DOUT = 64
G = 64
NC, NS = 2, 16
NW = NC * NS
LANES = 128
ECP = 2560
EPAD = ECP * LANES
NP = 10112
RPT = NP // NS
CPT = ECP // NW
HCPT = CPT // 2

_HIGH = lax.Precision.HIGHEST


@functools.cache
def _mesh():
    return plsc.VectorSubcoreMesh(core_axis_name="c", subcore_axis_name="s")


def _deg_body(dst_hbm, zeros_hbm, out_hbm, idx_v, ones_v, hist_sh,
              sem_i, sem_s):
    cid = lax.axis_index("c")
    sid = lax.axis_index("s")
    wid = sid * NC + cid
    cbase = wid * CPT

    pltpu.async_copy(dst_hbm.at[pl.ds(cbase, CPT)], idx_v, sem_i)

    def fill(i, carry):
        ones_v[i, :] = jnp.full((16,), 1.0, jnp.float32)
        return carry

    lax.fori_loop(0, LANES, fill, 0)
    pltpu.sync_copy(zeros_hbm.at[pl.ds(sid * RPT, RPT)],
                    hist_sh.at[pl.ds(sid * RPT, RPT)])
    pltpu.make_async_copy(dst_hbm.at[pl.ds(cbase, CPT)], idx_v, sem_i).wait()
    plsc.subcore_barrier()

    def body(j, carry):
        for b in range(8):
            pltpu.async_copy(ones_v, hist_sh.at[idx_v.at[8 * j + b]],
                             sem_s, add=True)
        for b in range(8):
            pltpu.make_async_copy(ones_v, hist_sh.at[idx_v.at[0]],
                                  sem_s).wait()
        return carry

    lax.fori_loop(0, CPT // 8, body, 0)
    plsc.subcore_barrier()
    pltpu.sync_copy(hist_sh.at[pl.ds(sid * RPT, RPT)],
                    out_hbm.at[cid, pl.ds(sid * RPT, RPT)])


@functools.cache
def _deg_call():
    return pl.kernel(
        _deg_body,
        out_type=jax.ShapeDtypeStruct((NC, NP, 16), jnp.float32),
        mesh=_mesh(),
        scratch_types=[
            pltpu.VMEM((CPT, LANES), jnp.int32),
            pltpu.VMEM((LANES, 16), jnp.float32),
            pltpu.VMEM_SHARED((NP, 16), jnp.float32),
            pltpu.SemaphoreType.DMA,
            pltpu.SemaphoreType.DMA,
        ],
    )


def _prop_body(hs_hbm, src_hbm, dst_hbm, zeros_hbm, out_hbm,
               sidx_v, didx_v, rows0, rows1, acc_sh,
               sem_i, sg0, sg1, ss0, ss1):
    cid = lax.axis_index("c")
    sid = lax.axis_index("s")
    wid = sid * NC + cid
    cbase = wid * CPT

    pltpu.async_copy(src_hbm.at[pl.ds(cbase, HCPT)], sidx_v, sem_i)
    pltpu.async_copy(dst_hbm.at[pl.ds(cbase, HCPT)], didx_v, sem_i)
    pltpu.sync_copy(zeros_hbm.at[pl.ds(sid * RPT, RPT)],
                    acc_sh.at[pl.ds(sid * RPT, RPT)])

    def gather(c, rows, sg):
        pltpu.async_copy(hs_hbm.at[sidx_v.at[c]], rows, sg)

    def gwait(rows, sg):
        pltpu.make_async_copy(hs_hbm.at[sidx_v.at[0]], rows, sg).wait()

    def scat(c, rows, ss):
        pltpu.async_copy(rows, acc_sh.at[didx_v.at[c]], ss, add=True)

    def swait(rows, ss):
        pltpu.make_async_copy(rows, acc_sh.at[didx_v.at[0]], ss).wait()

    def body(i, carry):
        c0 = 2 * i
        c1 = c0 + 1

        @pl.when(i > 0)
        def _():
            swait(rows0, ss0)

        gather(c0, rows0, sg0)

        @pl.when(i > 0)
        def _():
            swait(rows1, ss1)

        gather(c1, rows1, sg1)
        gwait(rows0, sg0)
        scat(c0, rows0, ss0)
        gwait(rows1, sg1)
        scat(c1, rows1, ss1)
        return carry

    for h in range(2):
        pltpu.make_async_copy(src_hbm.at[pl.ds(cbase, HCPT)], sidx_v,
                              sem_i).wait()
        pltpu.make_async_copy(dst_hbm.at[pl.ds(cbase, HCPT)], didx_v,
                              sem_i).wait()
        if h == 0:
            plsc.subcore_barrier()
        lax.fori_loop(0, HCPT // 2, body, 0)
        swait(rows0, ss0)
        swait(rows1, ss1)
        if h == 0:
            pltpu.async_copy(src_hbm.at[pl.ds(cbase + HCPT, HCPT)],
                             sidx_v, sem_i)
            pltpu.async_copy(dst_hbm.at[pl.ds(cbase + HCPT, HCPT)],
                             didx_v, sem_i)
    plsc.subcore_barrier()
    pltpu.sync_copy(acc_sh.at[pl.ds(sid * RPT, RPT)],
                    out_hbm.at[cid, pl.ds(sid * RPT, RPT)])


@functools.cache
def _prop_call():
    return pl.kernel(
        _prop_body,
        out_type=jax.ShapeDtypeStruct((NC, NP, DH), jnp.float32),
        mesh=_mesh(),
        scratch_types=[
            pltpu.VMEM((HCPT, LANES), jnp.int32),
            pltpu.VMEM((HCPT, LANES), jnp.int32),
            pltpu.VMEM((LANES, DH), jnp.float32),
            pltpu.VMEM((LANES, DH), jnp.float32),
            pltpu.VMEM_SHARED((NP, DH), jnp.float32),
            pltpu.SemaphoreType.DMA,
            pltpu.SemaphoreType.DMA,
            pltpu.SemaphoreType.DMA,
            pltpu.SemaphoreType.DMA,
            pltpu.SemaphoreType.DMA,
        ],
    )


def _prep_body(x_ref, w_ref, degp_ref, hs_ref, dinv_ref):
    deg = 1.0 + degp_ref[0, pl.ds(0, N), 0:1] + degp_ref[1, pl.ds(0, N), 0:1]
    dinv = lax.rsqrt(deg)
    h = jnp.dot(x_ref[...], w_ref[...], precision=_HIGH,
                preferred_element_type=jnp.float32)
    hs_ref[pl.ds(0, N), :] = h * dinv
    hs_ref[pl.ds(N, NP - N), :] = jnp.zeros((NP - N, DH), jnp.float32)
    dinv_ref[...] = dinv


def _mid_body(accp_ref, hs_ref, dinv_ref, b_ref, w_ref, out_ref):
    dinv = dinv_ref[...]
    agg = (accp_ref[0, pl.ds(0, N), :] + accp_ref[1, pl.ds(0, N), :]
           + hs_ref[pl.ds(0, N), :])
    z = jnp.maximum(dinv * agg + b_ref[...], 0.0)
    h2 = jnp.dot(z, w_ref[...], precision=_HIGH,
                 preferred_element_type=jnp.float32)
    out_ref[pl.ds(0, N), :] = h2 * dinv
    out_ref[pl.ds(N, NP - N), :] = jnp.zeros((NP - N, DH), jnp.float32)


def _fin_body(accp_ref, hs_ref, dinv_ref, b_ref, batch_ref, wfc_ref, bfc_ref,
              out_ref):
    dinv = dinv_ref[...]
    agg = (accp_ref[0, pl.ds(0, N), :] + accp_ref[1, pl.ds(0, N), :]
           + hs_ref[pl.ds(0, N), :])
    z = jnp.maximum(dinv * agg + b_ref[...], 0.0)
    gid = lax.broadcasted_iota(jnp.int32, (G, N), 0)
    onehot = (batch_ref[...] == gid).astype(jnp.float32)
    sums = jnp.dot(onehot, z, precision=_HIGH,
                   preferred_element_type=jnp.float32)
    cnts = jnp.sum(onehot, axis=1, keepdims=True)
    gpool = sums / jnp.maximum(cnts, 1.0)
    out_ref[...] = jnp.dot(gpool, wfc_ref[...], precision=_HIGH,
                           preferred_element_type=jnp.float32) + bfc_ref[...]


def _tc_call(body, out_shape):
    return pl.pallas_call(body, out_shape=out_shape)


def kernel(x, edge_index, batch, W1, b1, W2, b2, Wfc, bfc):
    src = edge_index[0].astype(jnp.int32)
    dst = edge_index[1].astype(jnp.int32)
    pad = jnp.full((EPAD - E,), N, jnp.int32)
    src2d = jnp.concatenate([src, pad]).reshape(ECP, LANES)
    dst2d = jnp.concatenate([dst, pad]).reshape(ECP, LANES)
    batch2d = batch.astype(jnp.int32).reshape(1, N)
    z128 = jnp.zeros((NP, DH), jnp.float32)
    z16 = jnp.zeros((NP, 16), jnp.float32)
    b1r = b1.reshape(1, DH)
    b2r = b2.reshape(1, DH)
    bfcr = bfc.reshape(1, DOUT)

    degp = _deg_call()(dst2d, z16)

    hs1, dinv = _tc_call(
        _prep_body,
        (jax.ShapeDtypeStruct((NP, DH), jnp.float32),
         jax.ShapeDtypeStruct((N, 1), jnp.float32)),
    )(x, W1, degp)

    acc1 = _prop_call()(hs1, src2d, dst2d, z128)

    hs2 = _tc_call(
        _mid_body, jax.ShapeDtypeStruct((NP, DH), jnp.float32),
    )(acc1, hs1, dinv, b1r, W2)

    acc2 = _prop_call()(hs2, src2d, dst2d, z128)

    out = _tc_call(
        _fin_body, jax.ShapeDtypeStruct((G, DOUT), jnp.float32),
    )(acc2, hs2, dinv, b2r, batch2d, Wfc, bfcr)
    return out


# speedup vs baseline: 25.9789x; 2.6167x over previous
import functools

import jax
import jax.numpy as jnp
from jax import lax
from jax.experimental import pallas as pl
from jax.experimental.pallas import tpu as pltpu
from jax.experimental.pallas import tpu_sc as plsc

N = 10000
E = 320000
DH = 128
DOUT = 64
G = 64
NC, NS = 2, 16
NW = NC * NS
LANES = 128
ECP = 2560
EPAD = ECP * LANES
NP = 10112
RPT = NP // NS
CPT = ECP // NW
HCPT = CPT // 2

_HIGH = lax.Precision.HIGHEST


@functools.cache
def _mesh():
    return plsc.VectorSubcoreMesh(core_axis_name="c", subcore_axis_name="s")


def _deg_body(dst_hbm, zeros_hbm, out_hbm, idx_v, ones_v, hist_sh,
              sem_i, sem_s):
    cid = lax.axis_index("c")
    sid = lax.axis_index("s")
    wid = sid * NC + cid
    cbase = wid * CPT

    pltpu.async_copy(dst_hbm.at[pl.ds(cbase, CPT)], idx_v, sem_i)

    def fill(i, carry):
        ones_v[i, :] = jnp.full((16,), 1.0, jnp.float32)
        return carry

    lax.fori_loop(0, LANES, fill, 0)
    pltpu.sync_copy(zeros_hbm.at[pl.ds(sid * RPT, RPT)],
                    hist_sh.at[pl.ds(sid * RPT, RPT)])
    pltpu.make_async_copy(dst_hbm.at[pl.ds(cbase, CPT)], idx_v, sem_i).wait()
    plsc.subcore_barrier()

    def body(j, carry):
        for b in range(8):
            pltpu.async_copy(ones_v, hist_sh.at[idx_v.at[8 * j + b]],
                             sem_s, add=True)
        for b in range(8):
            pltpu.make_async_copy(ones_v, hist_sh.at[idx_v.at[0]],
                                  sem_s).wait()
        return carry

    lax.fori_loop(0, CPT // 8, body, 0)
    plsc.subcore_barrier()
    pltpu.sync_copy(hist_sh.at[pl.ds(sid * RPT, RPT)],
                    out_hbm.at[cid, pl.ds(sid * RPT, RPT)])


@functools.cache
def _deg_call():
    return pl.kernel(
        _deg_body,
        out_type=jax.ShapeDtypeStruct((NC, NP, 16), jnp.float32),
        mesh=_mesh(),
        scratch_types=[
            pltpu.VMEM((CPT, LANES), jnp.int32),
            pltpu.VMEM((LANES, 16), jnp.float32),
            pltpu.VMEM_SHARED((NP, 16), jnp.float32),
            pltpu.SemaphoreType.DMA,
            pltpu.SemaphoreType.DMA,
        ],
    )


def _prop_body(hs_hbm, src_hbm, dst_hbm, zeros_hbm, out_hbm,
               sidx_v, didx_v, rows0, rows1, acc_sh,
               sem_i, sg0, sg1, ss0, ss1):
    cid = lax.axis_index("c")
    sid = lax.axis_index("s")
    wid = sid * NC + cid
    cbase = wid * CPT

    pltpu.async_copy(src_hbm.at[pl.ds(cbase, HCPT)], sidx_v, sem_i)
    pltpu.async_copy(dst_hbm.at[pl.ds(cbase, HCPT)], didx_v, sem_i)
    pltpu.sync_copy(zeros_hbm.at[pl.ds(sid * RPT, RPT)],
                    acc_sh.at[pl.ds(sid * RPT, RPT)])

    def gather(c, rows, sg):
        pltpu.async_copy(hs_hbm.at[sidx_v.at[c]], rows, sg)

    def gwait(rows, sg):
        pltpu.make_async_copy(hs_hbm.at[sidx_v.at[0]], rows, sg).wait()

    def scat(c, rows, ss):
        pltpu.async_copy(rows, acc_sh.at[didx_v.at[c]], ss, add=True)

    def swait(rows, ss):
        pltpu.make_async_copy(rows, acc_sh.at[didx_v.at[0]], ss).wait()

    def body(i, carry):
        c0 = 2 * i
        c1 = c0 + 1

        @pl.when(i > 0)
        def _():
            swait(rows0, ss0)

        gather(c0, rows0, sg0)

        @pl.when(i > 0)
        def _():
            swait(rows1, ss1)

        gather(c1, rows1, sg1)
        gwait(rows0, sg0)
        scat(c0, rows0, ss0)
        gwait(rows1, sg1)
        scat(c1, rows1, ss1)
        return carry

    for h in range(2):
        pltpu.make_async_copy(src_hbm.at[pl.ds(cbase, HCPT)], sidx_v,
                              sem_i).wait()
        pltpu.make_async_copy(dst_hbm.at[pl.ds(cbase, HCPT)], didx_v,
                              sem_i).wait()
        if h == 0:
            plsc.subcore_barrier()
        lax.fori_loop(0, HCPT // 2, body, 0)
        swait(rows0, ss0)
        swait(rows1, ss1)
        if h == 0:
            pltpu.async_copy(src_hbm.at[pl.ds(cbase + HCPT, HCPT)],
                             sidx_v, sem_i)
            pltpu.async_copy(dst_hbm.at[pl.ds(cbase + HCPT, HCPT)],
                             didx_v, sem_i)
    plsc.subcore_barrier()
    pltpu.sync_copy(acc_sh.at[pl.ds(sid * RPT, RPT)],
                    out_hbm.at[cid, pl.ds(sid * RPT, RPT)])


@functools.cache
def _prop_call():
    return pl.kernel(
        _prop_body,
        out_type=jax.ShapeDtypeStruct((NC, NP, DH), jnp.float32),
        mesh=_mesh(),
        scratch_types=[
            pltpu.VMEM((HCPT, LANES), jnp.int32),
            pltpu.VMEM((HCPT, LANES), jnp.int32),
            pltpu.VMEM((LANES, DH), jnp.float32),
            pltpu.VMEM((LANES, DH), jnp.float32),
            pltpu.VMEM_SHARED((NP, DH), jnp.float32),
            pltpu.SemaphoreType.DMA,
            pltpu.SemaphoreType.DMA,
            pltpu.SemaphoreType.DMA,
            pltpu.SemaphoreType.DMA,
            pltpu.SemaphoreType.DMA,
        ],
    )


def _prep_body(x_ref, w_ref, degp_ref, hs_ref, dinv_ref):
    deg = 1.0 + degp_ref[0, pl.ds(0, N), 0:1] + degp_ref[1, pl.ds(0, N), 0:1]
    dinv = lax.rsqrt(deg)
    h = jnp.dot(x_ref[...], w_ref[...], precision=_HIGH,
                preferred_element_type=jnp.float32)
    hs_ref[pl.ds(0, N), :] = h * dinv
    hs_ref[pl.ds(N, NP - N), :] = jnp.zeros((NP - N, DH), jnp.float32)
    dinv_ref[...] = dinv


def _mid_body(accp_ref, hs_ref, dinv_ref, b_ref, w_ref, out_ref):
    dinv = dinv_ref[...]
    agg = (accp_ref[0, pl.ds(0, N), :] + accp_ref[1, pl.ds(0, N), :]
           + hs_ref[pl.ds(0, N), :])
    z = jnp.maximum(dinv * agg + b_ref[...], 0.0)
    h2 = jnp.dot(z, w_ref[...], precision=_HIGH,
                 preferred_element_type=jnp.float32)
    out_ref[pl.ds(0, N), :] = h2 * dinv
    out_ref[pl.ds(N, NP - N), :] = jnp.zeros((NP - N, DH), jnp.float32)


def _fin_body(accp_ref, hs_ref, dinv_ref, b_ref, batch_ref, wfc_ref, bfc_ref,
              out_ref):
    dinv = dinv_ref[...]
    agg = (accp_ref[0, pl.ds(0, N), :] + accp_ref[1, pl.ds(0, N), :]
           + hs_ref[pl.ds(0, N), :])
    z = jnp.maximum(dinv * agg + b_ref[...], 0.0)
    gid = lax.broadcasted_iota(jnp.int32, (G, N), 0)
    onehot = (batch_ref[...] == gid).astype(jnp.float32)
    sums = jnp.dot(onehot, z, precision=_HIGH,
                   preferred_element_type=jnp.float32)
    cnts = jnp.sum(onehot, axis=1, keepdims=True)
    gpool = sums / jnp.maximum(cnts, 1.0)
    out_ref[...] = jnp.dot(gpool, wfc_ref[...], precision=_HIGH,
                           preferred_element_type=jnp.float32) + bfc_ref[...]


def _tc_call(body, out_shape):
    return pl.pallas_call(body, out_shape=out_shape)


def kernel(x, edge_index, batch, W1, b1, W2, b2, Wfc, bfc):
    src = edge_index[0].astype(jnp.int32)
    dst = edge_index[1].astype(jnp.int32)
    pad = N + jnp.arange(EPAD - E, dtype=jnp.int32) % (NP - N)
    src2d = jnp.concatenate([src, pad]).reshape(ECP, LANES)
    dst2d = jnp.concatenate([dst, pad]).reshape(ECP, LANES)
    batch2d = batch.astype(jnp.int32).reshape(1, N)
    z128 = jnp.zeros((NP, DH), jnp.float32)
    z16 = jnp.zeros((NP, 16), jnp.float32)
    b1r = b1.reshape(1, DH)
    b2r = b2.reshape(1, DH)
    bfcr = bfc.reshape(1, DOUT)

    degp = _deg_call()(dst2d, z16)

    hs1, dinv = _tc_call(
        _prep_body,
        (jax.ShapeDtypeStruct((NP, DH), jnp.float32),
         jax.ShapeDtypeStruct((N, 1), jnp.float32)),
    )(x, W1, degp)

    acc1 = _prop_call()(hs1, src2d, dst2d, z128)

    hs2 = _tc_call(
        _mid_body, jax.ShapeDtypeStruct((NP, DH), jnp.float32),
    )(acc1, hs1, dinv, b1r, W2)

    acc2 = _prop_call()(hs2, src2d, dst2d, z128)

    out = _tc_call(
        _fin_body, jax.ShapeDtypeStruct((G, DOUT), jnp.float32),
    )(acc2, hs2, dinv, b2r, batch2d, Wfc, bfcr)
    return out


# speedup vs baseline: 25.9822x; 1.0001x over previous
import functools

import jax
import jax.numpy as jnp
from jax import lax
from jax.experimental import pallas as pl
from jax.experimental.pallas import tpu as pltpu
from jax.experimental.pallas import tpu_sc as plsc

N = 10000
E = 320000
DH = 128
DOUT = 64
G = 64
NC, NS = 2, 16
NW = NC * NS
LANES = 128
ECP = 2560
EPAD = ECP * LANES
NP = 10112
RPT = NP // NS
CPT = ECP // NW
HCPT = CPT // 2

_HIGH = lax.Precision.HIGHEST


@functools.cache
def _mesh():
    return plsc.VectorSubcoreMesh(core_axis_name="c", subcore_axis_name="s")


def _deg_body(dst_hbm, zeros_hbm, out_hbm, idx_v, ones_v, hist_sh,
              sem_i, sem_s):
    cid = lax.axis_index("c")
    sid = lax.axis_index("s")
    wid = sid * NC + cid
    cbase = wid * CPT

    pltpu.async_copy(dst_hbm.at[pl.ds(cbase, CPT)], idx_v, sem_i)

    def fill(i, carry):
        ones_v[i, :] = jnp.full((16,), 1.0, jnp.float32)
        return carry

    lax.fori_loop(0, LANES, fill, 0)
    pltpu.sync_copy(zeros_hbm.at[pl.ds(sid * RPT, RPT)],
                    hist_sh.at[pl.ds(sid * RPT, RPT)])
    pltpu.make_async_copy(dst_hbm.at[pl.ds(cbase, CPT)], idx_v, sem_i).wait()
    plsc.subcore_barrier()

    def body(j, carry):
        for b in range(8):
            pltpu.async_copy(ones_v, hist_sh.at[idx_v.at[8 * j + b]],
                             sem_s, add=True)
        for b in range(8):
            pltpu.make_async_copy(ones_v, hist_sh.at[idx_v.at[0]],
                                  sem_s).wait()
        return carry

    lax.fori_loop(0, CPT // 8, body, 0)
    plsc.subcore_barrier()
    pltpu.sync_copy(hist_sh.at[pl.ds(sid * RPT, RPT)],
                    out_hbm.at[cid, pl.ds(sid * RPT, RPT)])


@functools.cache
def _deg_call():
    return pl.kernel(
        _deg_body,
        out_type=jax.ShapeDtypeStruct((NC, NP, 16), jnp.float32),
        mesh=_mesh(),
        scratch_types=[
            pltpu.VMEM((CPT, LANES), jnp.int32),
            pltpu.VMEM((LANES, 16), jnp.float32),
            pltpu.VMEM_SHARED((NP, 16), jnp.float32),
            pltpu.SemaphoreType.DMA,
            pltpu.SemaphoreType.DMA,
        ],
    )


def _prop_body(hs_hbm, src_hbm, dst_hbm, zeros_hbm, out_hbm,
               sidx_v, didx_v, rows0, rows1, acc_sh,
               sem_i, sg0, sg1, ss0, ss1):
    cid = lax.axis_index("c")
    sid = lax.axis_index("s")
    wid = sid * NC + cid
    cbase = wid * CPT

    pltpu.async_copy(src_hbm.at[pl.ds(cbase, HCPT)], sidx_v, sem_i)
    pltpu.async_copy(dst_hbm.at[pl.ds(cbase, HCPT)], didx_v, sem_i)
    pltpu.sync_copy(zeros_hbm.at[pl.ds(sid * RPT, RPT)],
                    acc_sh.at[pl.ds(sid * RPT, RPT)])

    def gather(c, rows, sg):
        pltpu.async_copy(hs_hbm.at[sidx_v.at[c]], rows, sg)

    def gwait(rows, sg):
        pltpu.make_async_copy(hs_hbm.at[sidx_v.at[0]], rows, sg).wait()

    def scat(c, rows, ss):
        pltpu.async_copy(rows, acc_sh.at[didx_v.at[c]], ss, add=True)

    def swait(rows, ss):
        pltpu.make_async_copy(rows, acc_sh.at[didx_v.at[0]], ss).wait()

    def body(i, carry):
        c0 = 2 * i
        c1 = c0 + 1

        @pl.when(i > 0)
        def _():
            swait(rows0, ss0)

        gather(c0, rows0, sg0)

        @pl.when(i > 0)
        def _():
            swait(rows1, ss1)

        gather(c1, rows1, sg1)
        gwait(rows0, sg0)
        scat(c0, rows0, ss0)
        gwait(rows1, sg1)
        scat(c1, rows1, ss1)
        return carry

    for h in range(2):
        pltpu.make_async_copy(src_hbm.at[pl.ds(cbase, HCPT)], sidx_v,
                              sem_i).wait()
        pltpu.make_async_copy(dst_hbm.at[pl.ds(cbase, HCPT)], didx_v,
                              sem_i).wait()
        if h == 0:
            plsc.subcore_barrier()
        lax.fori_loop(0, HCPT // 2, body, 0)
        swait(rows0, ss0)
        swait(rows1, ss1)
        if h == 0:
            pltpu.async_copy(src_hbm.at[pl.ds(cbase + HCPT, HCPT)],
                             sidx_v, sem_i)
            pltpu.async_copy(dst_hbm.at[pl.ds(cbase + HCPT, HCPT)],
                             didx_v, sem_i)
    plsc.subcore_barrier()
    pltpu.sync_copy(acc_sh.at[pl.ds(sid * RPT, RPT)],
                    out_hbm.at[cid, pl.ds(sid * RPT, RPT)])


@functools.cache
def _prop_call():
    return pl.kernel(
        _prop_body,
        out_type=jax.ShapeDtypeStruct((NC, NP, DH), jnp.float32),
        mesh=_mesh(),
        scratch_types=[
            pltpu.VMEM((HCPT, LANES), jnp.int32),
            pltpu.VMEM((HCPT, LANES), jnp.int32),
            pltpu.VMEM((LANES, DH), jnp.float32),
            pltpu.VMEM((LANES, DH), jnp.float32),
            pltpu.VMEM_SHARED((NP, DH), jnp.float32),
            pltpu.SemaphoreType.DMA,
            pltpu.SemaphoreType.DMA,
            pltpu.SemaphoreType.DMA,
            pltpu.SemaphoreType.DMA,
            pltpu.SemaphoreType.DMA,
        ],
    )


def _mm_body(x_ref, w_ref, h_ref):
    h_ref[...] = jnp.dot(x_ref[...], w_ref[...], precision=_HIGH,
                         preferred_element_type=jnp.float32)


def _prep_body(h_ref, degp_ref, hs_ref, dinv_ref):
    deg = 1.0 + degp_ref[0, pl.ds(0, N), 0:1] + degp_ref[1, pl.ds(0, N), 0:1]
    dinv = lax.rsqrt(deg)
    hs_ref[pl.ds(0, N), :] = h_ref[...] * dinv
    hs_ref[pl.ds(N, NP - N), :] = jnp.zeros((NP - N, DH), jnp.float32)
    dinv_ref[...] = dinv


def _mid_body(accp_ref, hs_ref, dinv_ref, b_ref, w_ref, out_ref):
    dinv = dinv_ref[...]
    agg = (accp_ref[0, pl.ds(0, N), :] + accp_ref[1, pl.ds(0, N), :]
           + hs_ref[pl.ds(0, N), :])
    z = jnp.maximum(dinv * agg + b_ref[...], 0.0)
    h2 = jnp.dot(z, w_ref[...], precision=_HIGH,
                 preferred_element_type=jnp.float32)
    out_ref[pl.ds(0, N), :] = h2 * dinv
    out_ref[pl.ds(N, NP - N), :] = jnp.zeros((NP - N, DH), jnp.float32)


def _fin_body(accp_ref, hs_ref, dinv_ref, b_ref, batch_ref, wfc_ref, bfc_ref,
              out_ref):
    dinv = dinv_ref[...]
    agg = (accp_ref[0, pl.ds(0, N), :] + accp_ref[1, pl.ds(0, N), :]
           + hs_ref[pl.ds(0, N), :])
    z = jnp.maximum(dinv * agg + b_ref[...], 0.0)
    gid = lax.broadcasted_iota(jnp.int32, (G, N), 0)
    onehot = (batch_ref[...] == gid).astype(jnp.float32)
    sums = jnp.dot(onehot, z, precision=_HIGH,
                   preferred_element_type=jnp.float32)
    cnts = jnp.sum(onehot, axis=1, keepdims=True)
    gpool = sums / jnp.maximum(cnts, 1.0)
    out_ref[...] = jnp.dot(gpool, wfc_ref[...], precision=_HIGH,
                           preferred_element_type=jnp.float32) + bfc_ref[...]


def _tc_call(body, out_shape):
    return pl.pallas_call(body, out_shape=out_shape)


def kernel(x, edge_index, batch, W1, b1, W2, b2, Wfc, bfc):
    src = edge_index[0].astype(jnp.int32)
    dst = edge_index[1].astype(jnp.int32)
    pad = N + jnp.arange(EPAD - E, dtype=jnp.int32) % (NP - N)
    src2d = jnp.concatenate([src, pad]).reshape(ECP, LANES)
    dst2d = jnp.concatenate([dst, pad]).reshape(ECP, LANES)
    batch2d = batch.astype(jnp.int32).reshape(1, N)
    z128 = jnp.zeros((NP, DH), jnp.float32)
    z16 = jnp.zeros((NP, 16), jnp.float32)
    b1r = b1.reshape(1, DH)
    b2r = b2.reshape(1, DH)
    bfcr = bfc.reshape(1, DOUT)

    degp = _deg_call()(dst2d, z16)

    h1 = _tc_call(
        _mm_body, jax.ShapeDtypeStruct((N, DH), jnp.float32),
    )(x, W1)

    hs1, dinv = _tc_call(
        _prep_body,
        (jax.ShapeDtypeStruct((NP, DH), jnp.float32),
         jax.ShapeDtypeStruct((N, 1), jnp.float32)),
    )(h1, degp)

    acc1 = _prop_call()(hs1, src2d, dst2d, z128)

    hs2 = _tc_call(
        _mid_body, jax.ShapeDtypeStruct((NP, DH), jnp.float32),
    )(acc1, hs1, dinv, b1r, W2)

    acc2 = _prop_call()(hs2, src2d, dst2d, z128)

    out = _tc_call(
        _fin_body, jax.ShapeDtypeStruct((G, DOUT), jnp.float32),
    )(acc2, hs2, dinv, b2r, batch2d, Wfc, bfcr)
    return out


# speedup vs baseline: 28.4985x; 1.0968x over previous
import functools

import jax
import jax.numpy as jnp
from jax import lax
from jax.experimental import pallas as pl
from jax.experimental.pallas import tpu as pltpu
from jax.experimental.pallas import tpu_sc as plsc

N = 10000
E = 320000
DH = 128
DOUT = 64
G = 64
NC, NS = 2, 16
NW = NC * NS
LANES = 128
ECP = 2560
EPAD = ECP * LANES
NP = 10112
RPT = NP // NS
CPT = ECP // NW
DHALF = DH // 2
CPS = ECP // NS
NPAIR = CPS // 2

_HIGH = lax.Precision.HIGHEST


@functools.cache
def _mesh():
    return plsc.VectorSubcoreMesh(core_axis_name="c", subcore_axis_name="s")


def _deg_body(dst_hbm, zeros_hbm, out_hbm, idx_v, ones_v, hist_sh,
              sem_i, sem_s):
    cid = lax.axis_index("c")
    sid = lax.axis_index("s")
    wid = sid * NC + cid
    cbase = wid * CPT

    pltpu.async_copy(dst_hbm.at[pl.ds(cbase, CPT)], idx_v, sem_i)

    def fill(i, carry):
        ones_v[i, :] = jnp.full((16,), 1.0, jnp.float32)
        return carry

    lax.fori_loop(0, LANES, fill, 0)
    pltpu.sync_copy(zeros_hbm.at[pl.ds(sid * RPT, RPT)],
                    hist_sh.at[pl.ds(sid * RPT, RPT)])
    pltpu.make_async_copy(dst_hbm.at[pl.ds(cbase, CPT)], idx_v, sem_i).wait()
    plsc.subcore_barrier()

    def body(j, carry):
        for b in range(8):
            pltpu.async_copy(ones_v, hist_sh.at[idx_v.at[8 * j + b]],
                             sem_s, add=True)
        for b in range(8):
            pltpu.make_async_copy(ones_v, hist_sh.at[idx_v.at[0]],
                                  sem_s).wait()
        return carry

    lax.fori_loop(0, CPT // 8, body, 0)
    plsc.subcore_barrier()
    pltpu.sync_copy(hist_sh.at[pl.ds(sid * RPT, RPT)],
                    out_hbm.at[cid, pl.ds(sid * RPT, RPT)])


@functools.cache
def _deg_call():
    return pl.kernel(
        _deg_body,
        out_type=jax.ShapeDtypeStruct((NC, NP, 16), jnp.float32),
        mesh=_mesh(),
        scratch_types=[
            pltpu.VMEM((CPT, LANES), jnp.int32),
            pltpu.VMEM((LANES, 16), jnp.float32),
            pltpu.VMEM_SHARED((NP, 16), jnp.float32),
            pltpu.SemaphoreType.DMA,
            pltpu.SemaphoreType.DMA,
        ],
    )


def _prop_body(hs0_hbm, hs1_hbm, src_hbm, dst_hbm, zeros_hbm, out_hbm,
               sidx_v, didx_v, rows0, rows1, rows2, rows3, acc_sh,
               sem_i, sg0, sg1, sg2, sg3, ss0, ss1, ss2, ss3):
    cid = lax.axis_index("c")
    sid = lax.axis_index("s")
    cbase = sid * CPS

    pltpu.async_copy(src_hbm.at[pl.ds(cbase, CPS)], sidx_v, sem_i)
    pltpu.async_copy(dst_hbm.at[pl.ds(cbase, CPS)], didx_v, sem_i)
    pltpu.sync_copy(zeros_hbm.at[pl.ds(sid * RPT, RPT)],
                    acc_sh.at[pl.ds(sid * RPT, RPT)])
    pltpu.make_async_copy(src_hbm.at[pl.ds(cbase, CPS)], sidx_v, sem_i).wait()
    pltpu.make_async_copy(dst_hbm.at[pl.ds(cbase, CPS)], didx_v, sem_i).wait()
    plsc.subcore_barrier()

    def gather(c, rows, sg):
        idx = sidx_v.at[c]

        @pl.when(cid == 0)
        def _():
            pltpu.async_copy(hs0_hbm.at[idx], rows, sg)

        @pl.when(cid == 1)
        def _():
            pltpu.async_copy(hs1_hbm.at[idx], rows, sg)

    def gwait(rows, sg):
        pltpu.make_async_copy(hs0_hbm.at[sidx_v.at[0]], rows, sg).wait()

    def scat(c, rows, ss):
        pltpu.async_copy(rows, acc_sh.at[didx_v.at[c]], ss, add=True)

    def swait(rows, ss):
        pltpu.make_async_copy(rows, acc_sh.at[didx_v.at[0]], ss).wait()

    bufs = ((rows0, sg0, ss0), (rows1, sg1, ss1),
            (rows2, sg2, ss2), (rows3, sg3, ss3))

    def body(i, carry):
        for b, (rows, sg, ss) in enumerate(bufs):

            @pl.when(i > 0)
            def _(rows=rows, ss=ss):
                swait(rows, ss)

            gather(4 * i + b, rows, sg)
        for b, (rows, sg, ss) in enumerate(bufs):
            gwait(rows, sg)
            scat(4 * i + b, rows, ss)
        return carry

    lax.fori_loop(0, CPS // 4, body, 0)
    for rows, sg, ss in bufs:
        swait(rows, ss)
    plsc.subcore_barrier()
    pltpu.sync_copy(acc_sh.at[pl.ds(sid * RPT, RPT)],
                    out_hbm.at[cid, pl.ds(sid * RPT, RPT)])


@functools.cache
def _prop_call():
    return pl.kernel(
        _prop_body,
        out_type=jax.ShapeDtypeStruct((NC, NP, DHALF), jnp.float32),
        mesh=_mesh(),
        compiler_params=pltpu.CompilerParams(use_tc_tiling_on_sc=False),
        scratch_types=[
            pltpu.VMEM((CPS, LANES), jnp.int32),
            pltpu.VMEM((CPS, LANES), jnp.int32),
            pltpu.VMEM((LANES, DHALF), jnp.float32),
            pltpu.VMEM((LANES, DHALF), jnp.float32),
            pltpu.VMEM((LANES, DHALF), jnp.float32),
            pltpu.VMEM((LANES, DHALF), jnp.float32),
            pltpu.VMEM_SHARED((NP, DHALF), jnp.float32),
            pltpu.SemaphoreType.DMA,
            pltpu.SemaphoreType.DMA,
            pltpu.SemaphoreType.DMA,
            pltpu.SemaphoreType.DMA,
            pltpu.SemaphoreType.DMA,
            pltpu.SemaphoreType.DMA,
            pltpu.SemaphoreType.DMA,
            pltpu.SemaphoreType.DMA,
            pltpu.SemaphoreType.DMA,
        ],
    )


def _mm_body(x_ref, w_ref, h_ref):
    h_ref[pl.ds(0, N), :] = jnp.dot(x_ref[...], w_ref[...], precision=_HIGH,
                                    preferred_element_type=jnp.float32)
    h_ref[pl.ds(N, NP - N), :] = jnp.zeros((NP - N, DH), jnp.float32)


def _prep_body(h_ref, degp_ref, hsa_ref, hsb_ref, dinv_ref):
    deg = 1.0 + degp_ref[0, :, 0:1] + degp_ref[1, :, 0:1]
    dinv = lax.rsqrt(deg)
    vals = h_ref[...] * dinv
    hsa_ref[...] = vals[:, 0:DHALF]
    hsb_ref[...] = vals[:, DHALF:DH]
    dinv_ref[...] = dinv


def _mid_body(accp_ref, hsa_ref, hsb_ref, dinv_ref, b_ref, w_ref,
              outa_ref, outb_ref):
    dinv = dinv_ref[...]
    za = jnp.maximum(dinv * (accp_ref[0] + hsa_ref[...])
                     + b_ref[0:1, pl.ds(0, DHALF)], 0.0)
    zb = jnp.maximum(dinv * (accp_ref[1] + hsb_ref[...])
                     + b_ref[0:1, pl.ds(DHALF, DHALF)], 0.0)
    h2 = (jnp.dot(za, w_ref[pl.ds(0, DHALF), :], precision=_HIGH,
                  preferred_element_type=jnp.float32)
          + jnp.dot(zb, w_ref[pl.ds(DHALF, DHALF), :], precision=_HIGH,
                    preferred_element_type=jnp.float32))
    vals = h2 * dinv
    outa_ref[...] = vals[:, 0:DHALF]
    outb_ref[...] = vals[:, DHALF:DH]


def _half_relu(accp_ref, hs_ref, dinv, b_ref, half):
    agg = accp_ref[half, pl.ds(0, N), :] + hs_ref[pl.ds(0, N), :]
    bias = b_ref[0:1, pl.ds(half * DHALF, DHALF)]
    return jnp.maximum(dinv * agg + bias, 0.0)


def _fin_body(accp_ref, hsa_ref, hsb_ref, dinv_ref, b_ref, batch_ref,
              wfc_ref, bfc_ref, out_ref):
    dinv = dinv_ref[pl.ds(0, N), :]
    za = _half_relu(accp_ref, hsa_ref, dinv, b_ref, 0)
    zb = _half_relu(accp_ref, hsb_ref, dinv, b_ref, 1)
    gid = lax.broadcasted_iota(jnp.int32, (G, N), 0)
    onehot = (batch_ref[...] == gid).astype(jnp.float32)
    sumsa = jnp.dot(onehot, za, precision=_HIGH,
                    preferred_element_type=jnp.float32)
    sumsb = jnp.dot(onehot, zb, precision=_HIGH,
                    preferred_element_type=jnp.float32)
    cnts = jnp.maximum(jnp.sum(onehot, axis=1, keepdims=True), 1.0)
    out_ref[...] = (jnp.dot(sumsa / cnts, wfc_ref[pl.ds(0, DHALF), :],
                            precision=_HIGH,
                            preferred_element_type=jnp.float32)
                    + jnp.dot(sumsb / cnts, wfc_ref[pl.ds(DHALF, DHALF), :],
                              precision=_HIGH,
                              preferred_element_type=jnp.float32)
                    + bfc_ref[...])


def _tc_call(body, out_shape):
    return pl.pallas_call(body, out_shape=out_shape)


MBLK = NP // 8


@functools.cache
def _mid_call():
    half = jax.ShapeDtypeStruct((NP, DHALF), jnp.float32)
    return pl.pallas_call(
        _mid_body,
        grid=(NP // MBLK,),
        in_specs=[
            pl.BlockSpec((2, MBLK, DHALF), lambda i: (0, i, 0)),
            pl.BlockSpec((MBLK, DHALF), lambda i: (i, 0)),
            pl.BlockSpec((MBLK, DHALF), lambda i: (i, 0)),
            pl.BlockSpec((MBLK, 1), lambda i: (i, 0)),
            pl.BlockSpec((1, DH), lambda i: (0, 0)),
            pl.BlockSpec((DH, DH), lambda i: (0, 0)),
        ],
        out_specs=[
            pl.BlockSpec((MBLK, DHALF), lambda i: (i, 0)),
            pl.BlockSpec((MBLK, DHALF), lambda i: (i, 0)),
        ],
        out_shape=(half, half),
    )


def kernel(x, edge_index, batch, W1, b1, W2, b2, Wfc, bfc):
    src = edge_index[0].astype(jnp.int32)
    dst = edge_index[1].astype(jnp.int32)
    pad = N + jnp.arange(EPAD - E, dtype=jnp.int32) % (NP - N)
    src2d = jnp.concatenate([src, pad]).reshape(ECP, LANES)
    dst2d = jnp.concatenate([dst, pad]).reshape(ECP, LANES)
    batch2d = batch.astype(jnp.int32).reshape(1, N)
    z64 = jnp.zeros((NP, DHALF), jnp.float32)
    z16 = jnp.zeros((NP, 16), jnp.float32)
    b1r = b1.reshape(1, DH)
    b2r = b2.reshape(1, DH)
    bfcr = bfc.reshape(1, DOUT)

    degp = _deg_call()(dst2d, z16)

    h1 = _tc_call(
        _mm_body, jax.ShapeDtypeStruct((NP, DH), jnp.float32),
    )(x, W1)

    hs1a, hs1b, dinv = _tc_call(
        _prep_body,
        (jax.ShapeDtypeStruct((NP, DHALF), jnp.float32),
         jax.ShapeDtypeStruct((NP, DHALF), jnp.float32),
         jax.ShapeDtypeStruct((NP, 1), jnp.float32)),
    )(h1, degp)

    acc1 = _prop_call()(hs1a, hs1b, src2d, dst2d, z64)

    hs2a, hs2b = _mid_call()(acc1, hs1a, hs1b, dinv, b1r, W2)

    acc2 = _prop_call()(hs2a, hs2b, src2d, dst2d, z64)

    out = _tc_call(
        _fin_body, jax.ShapeDtypeStruct((G, DOUT), jnp.float32),
    )(acc2, hs2a, hs2b, dinv, b2r, batch2d, Wfc, bfcr)
    return out
